# baseline (speedup 1.0000x reference)
import jax
import jax.numpy as jnp
from jax.experimental import pallas as pl
from jax.experimental.pallas import tpu as pltpu

N = 4096
D = 128
T = 512
NB = N // T


def _heavy_body(s_ref, se_ref, init_ref, adj_ref, out_ref, oexp_ref):
    i = pl.program_id(0)
    j = pl.program_id(1)

    @pl.when(jnp.logical_and(i == 0, j == 0))
    def _():
        out_ref[...] = init_ref[...]

    adj_t = adj_ref[...]
    se_j = se_ref[pl.ds(j * 2 * T, 2 * T), :]
    out_ref[pl.ds(i * T, T), :] += jnp.dot(
        adj_t, se_j, preferred_element_type=jnp.float32)

    s_i = s_ref[pl.ds(i * T, T), :]
    c = jax.lax.dot_general(
        adj_t, s_i, (((0,), (0,)), ((), ())),
        preferred_element_type=jnp.float32)

    @pl.when(i == 0)
    def _():
        oexp_ref[pl.ds(j * 2 * T, 2 * T), :] = c

    @pl.when(i > 0)
    def _():
        oexp_ref[pl.ds(j * 2 * T, 2 * T), :] += c


def _heavy(adj2, s, se, init):
    return pl.pallas_call(
        _heavy_body,
        grid=(NB, NB),
        in_specs=[
            pl.BlockSpec((N, D), lambda i, j: (0, 0)),
            pl.BlockSpec((2 * N, D), lambda i, j: (0, 0)),
            pl.BlockSpec((N, D), lambda i, j: (0, 0)),
            pl.BlockSpec((T, 2 * T), lambda i, j: (i, j)),
        ],
        out_specs=[
            pl.BlockSpec((N, D), lambda i, j: (0, 0)),
            pl.BlockSpec((2 * N, D), lambda i, j: (0, 0)),
        ],
        out_shape=[
            jax.ShapeDtypeStruct((N, D), jnp.float32),
            jax.ShapeDtypeStruct((2 * N, D), jnp.float32),
        ],
        compiler_params=pltpu.CompilerParams(
            dimension_semantics=("arbitrary", "arbitrary")),
    )(s, se, init, adj2)


def _mm_body(x_ref, w_ref, o_ref):
    o_ref[...] = jnp.dot(x_ref[...], w_ref[...],
                         preferred_element_type=jnp.float32)


def _mm(x, w):
    return pl.pallas_call(
        _mm_body,
        out_shape=jax.ShapeDtypeStruct((N, D), jnp.float32),
    )(x, w)


def _mid_body(rw_ref, hfwd_ref, hexp_ref, w1_ref, b1_ref, s1_ref, init2_ref):
    rw0 = rw_ref[0, 0]
    rw1 = rw_ref[1, 0]
    out0 = hfwd_ref[...] + rw0 * hexp_ref[:, 0, :] + rw1 * hexp_ref[:, 1, :]
    s1_ref[...] = 0.5 * jnp.dot(out0, w1_ref[...],
                                preferred_element_type=jnp.float32)
    init2_ref[...] = 0.5 * out0 + 0.5 * b1_ref[...]


def _mid(rw, hfwd, hexp3, W1, b1):
    return pl.pallas_call(
        _mid_body,
        out_shape=[
            jax.ShapeDtypeStruct((N, D), jnp.float32),
            jax.ShapeDtypeStruct((N, D), jnp.float32),
        ],
    )(rw, hfwd, hexp3, W1, b1)


def _fin_body(rw_ref, hfwd_ref, hexp_ref, o_ref):
    rw0 = rw_ref[0, 0]
    rw1 = rw_ref[1, 0]
    o_ref[...] = (hfwd_ref[...]
                  + rw0 * hexp_ref[:, 0, :] + rw1 * hexp_ref[:, 1, :])


def _fin(rw, hfwd, hexp3):
    return pl.pallas_call(
        _fin_body,
        out_shape=jax.ShapeDtypeStruct((N, D), jnp.float32),
    )(rw, hfwd, hexp3)


def kernel(x_feature, all_adj_matrix, W0, b0, W1, b1, relation_weight):
    adj2 = all_adj_matrix.reshape(N, 2 * N)
    rwf = relation_weight.reshape(2)

    s0 = _mm(x_feature, W0)
    s0e = (s0[:, None, :] * rwf[None, :, None]).reshape(2 * N, D)
    init1 = jnp.broadcast_to(b0[None, :], (N, D))
    hfwd0, hexp0 = _heavy(adj2, s0, s0e, init1)

    s1, init2 = _mid(relation_weight, hfwd0, hexp0.reshape(N, 2, D), W1,
                     b1.reshape(1, D))
    s1e = (s1[:, None, :] * rwf[None, :, None]).reshape(2 * N, D)
    hfwd1, hexp1 = _heavy(adj2, s1, s1e, init2)

    return _fin(relation_weight, hfwd1, hexp1.reshape(N, 2, D))
